# baseline (speedup 1.0000x reference)
import functools

import jax
import jax.numpy as jnp
from jax.experimental import pallas as pl
from jax.experimental.pallas import tpu as pltpu
from jax.experimental.pallas import tpu_sc as plsc


def _sc_gather(table, inds_flat):
    n = inds_flat.shape[0]
    d = table.shape[1]
    nc, ns = 2, 16
    nw = nc * ns
    b_per_w = n // nw
    ch = 800
    n_ch = b_per_w // ch
    mesh = plsc.VectorSubcoreMesh(core_axis_name="c", subcore_axis_name="s")

    @functools.partial(
        pl.kernel, mesh=mesh,
        out_type=jax.ShapeDtypeStruct((n, d), table.dtype),
        compiler_params=pltpu.CompilerParams(use_tc_tiling_on_sc=False),
        scratch_types=[
            pltpu.VMEM((ch,), jnp.int32),
            pltpu.VMEM((ch, d), jnp.float32),
            pltpu.SemaphoreType.DMA,
        ],
    )
    def k(table_hbm, idx_hbm, out_hbm, idx_v, rows_v, sem):
        wid = jax.lax.axis_index("s") * nc + jax.lax.axis_index("c")
        base = wid * b_per_w

        @pl.loop(0, n_ch)
        def _(c):
            off = base + c * ch
            pltpu.sync_copy(idx_hbm.at[pl.ds(off, ch)], idx_v)
            pltpu.async_copy(table_hbm.at[idx_v], rows_v, sem).wait()
            pltpu.sync_copy(rows_v, out_hbm.at[pl.ds(off, ch)])

    return k(table, inds_flat)


def _logits_body(em_ref, mpw_ref, wa_ref, va_ref, out_ref):
    wc = jax.lax.dot(mpw_ref[...], wa_ref[...],
                     preferred_element_type=jnp.float32)
    t = jnp.tanh(jax.lax.dot(em_ref[...], wc,
                             preferred_element_type=jnp.float32))
    out_ref[...] = jax.lax.dot(t, va_ref[...],
                               preferred_element_type=jnp.float32)


def _tc_logits(em, mp_w, W_a, v_a):
    n, d = em.shape
    blk = 12800
    return pl.pallas_call(
        _logits_body,
        grid=(n // blk,),
        in_specs=[
            pl.BlockSpec((blk, d), lambda i: (i, 0)),
            pl.BlockSpec((d, d), lambda i: (0, 0)),
            pl.BlockSpec((d, W_a.shape[1]), lambda i: (0, 0)),
            pl.BlockSpec((W_a.shape[1], 1), lambda i: (0, 0)),
        ],
        out_specs=pl.BlockSpec((blk, 1), lambda i: (i, 0)),
        out_shape=jax.ShapeDtypeStruct((n, 1), jnp.float32),
    )(em, mp_w, W_a, v_a.reshape(-1, 1))


def _score_body(em_ref, g_ref, m_ref, mpw_ref, out_ref):
    bb, ll, d = em_ref.shape
    logits = g_ref[...]
    mx = jnp.max(logits, axis=1, keepdims=True)
    e = jnp.exp(logits - mx)
    alpha = e / jnp.sum(e, axis=1, keepdims=True)

    mpw = mpw_ref[...]
    gram = jax.lax.dot(mpw, mpw.T, preferred_element_type=jnp.float32)
    em0 = em_ref[:, 0, :]
    q = jax.lax.dot(em0, gram, preferred_element_type=jnp.float32)

    x3 = em_ref[...]
    dots = jnp.sum(x3 * q[:, None, :], axis=2)

    lidx = jax.lax.broadcasted_iota(jnp.int32, (bb, ll), 1)
    mm = jnp.where(lidx > 0, m_ref[...], 0.0)
    num = jnp.sum(alpha * dots * mm, axis=1, keepdims=True) * alpha[:, 0:1]
    den = jnp.sum(mm, axis=1, keepdims=True) + 1e-8
    out_ref[...] = num / den


def _tc_score(em3, g, mask, mp_w):
    b, ll, d = em3.shape
    bb = 256
    return pl.pallas_call(
        _score_body,
        grid=(b // bb,),
        in_specs=[
            pl.BlockSpec((bb, ll, d), lambda i: (i, 0, 0)),
            pl.BlockSpec((bb, ll), lambda i: (i, 0)),
            pl.BlockSpec((bb, ll), lambda i: (i, 0)),
            pl.BlockSpec((d, d), lambda i: (0, 0)),
        ],
        out_specs=pl.BlockSpec((bb, 1), lambda i: (i, 0)),
        out_shape=jax.ShapeDtypeStruct((b, 1), jnp.float32),
    )(em3, g, mask, mp_w)


def kernel(inds, mask, table, mp_w, W_a, v_a):
    b, ll = inds.shape
    d = table.shape[1]
    em = _sc_gather(table, inds.reshape(-1).astype(jnp.int32))
    g = _tc_logits(em, mp_w, W_a, v_a).reshape(b, ll)
    scores = _tc_score(em.reshape(b, ll, d), g, mask, mp_w)
    return scores.reshape(b)


# speedup vs baseline: 1.0030x; 1.0030x over previous
import functools

import jax
import jax.numpy as jnp
from jax.experimental import pallas as pl
from jax.experimental.pallas import tpu as pltpu
from jax.experimental.pallas import tpu_sc as plsc


def _sc_gather(table2, idx2):
    n = idx2.shape[0]
    d = table2.shape[1]
    nc, ns = 2, 16
    nw = nc * ns
    b_per_w = n // nw
    ch = 800
    n_ch = b_per_w // ch
    mesh = plsc.VectorSubcoreMesh(core_axis_name="c", subcore_axis_name="s")

    @functools.partial(
        pl.kernel, mesh=mesh,
        out_type=jax.ShapeDtypeStruct((n, d), table2.dtype),
        scratch_types=[
            pltpu.VMEM((ch,), jnp.int32),
            pltpu.VMEM((ch, d), jnp.float32),
            pltpu.SemaphoreType.DMA,
        ],
    )
    def k(table_hbm, idx_hbm, out_hbm, idx_v, rows_v, sem):
        wid = jax.lax.axis_index("s") * nc + jax.lax.axis_index("c")
        base = wid * b_per_w

        @pl.loop(0, n_ch)
        def _(c):
            off = base + c * ch
            pltpu.sync_copy(idx_hbm.at[pl.ds(off, ch)], idx_v)
            pltpu.async_copy(table_hbm.at[idx_v], rows_v, sem).wait()
            pltpu.sync_copy(rows_v, out_hbm.at[pl.ds(off, ch)])

    return k(table2, idx2)


def _fused_body(x_ref, p_ref, m_ref, mpw_ref, wa_ref, va_ref, out_ref):
    bb, ll = p_ref.shape
    d = x_ref.shape[1] // 2

    x = x_ref[...]
    xl = x[:, :d]
    xr = x[:, d:]
    psel = p_ref[...] > 0

    mpw = mpw_ref[...]
    wc = jax.lax.dot(mpw, wa_ref[...], preferred_element_type=jnp.float32)
    va = va_ref[...]
    tl = jnp.tanh(jax.lax.dot(xl, wc, preferred_element_type=jnp.float32))
    tr = jnp.tanh(jax.lax.dot(xr, wc, preferred_element_type=jnp.float32))
    gl = jax.lax.dot(tl, va, preferred_element_type=jnp.float32)
    gr = jax.lax.dot(tr, va, preferred_element_type=jnp.float32)
    g = jnp.where(psel, gr.reshape(bb, ll), gl.reshape(bb, ll))

    mx = jnp.max(g, axis=1, keepdims=True)
    e = jnp.exp(g - mx)
    alpha = e / jnp.sum(e, axis=1, keepdims=True)

    em3l = xl.reshape(bb, ll, d)
    em3r = xr.reshape(bb, ll, d)
    em0 = jnp.where(psel[:, 0:1], em3r[:, 0, :], em3l[:, 0, :])
    gram = jax.lax.dot(mpw, mpw.T, preferred_element_type=jnp.float32)
    q = jax.lax.dot(em0, gram, preferred_element_type=jnp.float32)
    dl = jnp.sum(em3l * q[:, None, :], axis=2)
    dr = jnp.sum(em3r * q[:, None, :], axis=2)
    dots = jnp.where(psel, dr, dl)

    lidx = jax.lax.broadcasted_iota(jnp.int32, (bb, ll), 1)
    mm = jnp.where(lidx > 0, m_ref[...], 0.0)
    num = jnp.sum(alpha * dots * mm, axis=1, keepdims=True) * alpha[:, 0:1]
    den = jnp.sum(mm, axis=1, keepdims=True) + 1e-8
    out_ref[...] = num / den


def _tc_fused(x128, par2, mask, mp_w, W_a, v_a):
    b, ll = par2.shape
    d = mp_w.shape[0]
    bb = 256
    flat = bb * ll
    return pl.pallas_call(
        _fused_body,
        grid=(b // bb,),
        in_specs=[
            pl.BlockSpec((flat, 2 * d), lambda i: (i, 0)),
            pl.BlockSpec((bb, ll), lambda i: (i, 0)),
            pl.BlockSpec((bb, ll), lambda i: (i, 0)),
            pl.BlockSpec((d, d), lambda i: (0, 0)),
            pl.BlockSpec((d, W_a.shape[1]), lambda i: (0, 0)),
            pl.BlockSpec((W_a.shape[1], 1), lambda i: (0, 0)),
        ],
        out_specs=pl.BlockSpec((bb, 1), lambda i: (i, 0)),
        out_shape=jax.ShapeDtypeStruct((b, 1), jnp.float32),
    )(x128, par2, mask, mp_w, W_a, v_a.reshape(-1, 1))


def kernel(inds, mask, table, mp_w, W_a, v_a):
    b, ll = inds.shape
    v, d = table.shape
    inds32 = inds.astype(jnp.int32)
    table2 = table.reshape(v // 2, 2 * d)
    x128 = _sc_gather(table2, (inds32 >> 1).reshape(-1))
    scores = _tc_fused(x128, inds32 & 1, mask, mp_w, W_a, v_a)
    return scores.reshape(b)


# speedup vs baseline: 1.2542x; 1.2505x over previous
import functools

import jax
import jax.numpy as jnp
from jax.experimental import pallas as pl
from jax.experimental.pallas import tpu as pltpu
from jax.experimental.pallas import tpu_sc as plsc


def _pair_body(t_ref, out_ref):
    x = t_ref[...]
    w = x.shape[1]
    y = jnp.transpose(x)
    v = y.reshape(w // 2, 2, x.shape[0])
    out_ref[...] = jnp.concatenate([v[:, 0, :], v[:, 1, :]], axis=1)


def _tc_pair_table(tt):
    d, v = tt.shape
    w = 12800
    grid = (v + w - 1) // w
    return pl.pallas_call(
        _pair_body,
        grid=(grid,),
        in_specs=[pl.BlockSpec((d, w), lambda i: (0, i))],
        out_specs=pl.BlockSpec((w // 2, 2 * d), lambda i: (i, 0)),
        out_shape=jax.ShapeDtypeStruct((v // 2, 2 * d), jnp.float32),
    )(tt)


def _sc_gather(table2, idx2):
    n = idx2.shape[0]
    d = table2.shape[1]
    nc, ns = 2, 16
    nw = nc * ns
    b_per_w = n // nw
    ch = 800
    n_ch = b_per_w // ch
    mesh = plsc.VectorSubcoreMesh(core_axis_name="c", subcore_axis_name="s")

    @functools.partial(
        pl.kernel, mesh=mesh,
        out_type=jax.ShapeDtypeStruct((n, d), jnp.float32),
        scratch_types=[
            pltpu.VMEM((ch,), jnp.int32),
            pltpu.VMEM((ch, d), jnp.float32),
            pltpu.SemaphoreType.DMA,
        ],
    )
    def k(table_hbm, idx_hbm, out_hbm, idx_v, rows_v, sem):
        wid = jax.lax.axis_index("s") * nc + jax.lax.axis_index("c")
        base = wid * b_per_w

        @pl.loop(0, n_ch)
        def _(c):
            off = base + c * ch
            pltpu.sync_copy(idx_hbm.at[pl.ds(off, ch)], idx_v)
            pltpu.async_copy(table_hbm.at[idx_v], rows_v, sem).wait()
            pltpu.sync_copy(rows_v, out_hbm.at[pl.ds(off, ch)])

    return k(table2, idx2)


def _fused_body(x_ref, p_ref, m_ref, mpw_ref, wa_ref, va_ref, out_ref):
    bb, ll = p_ref.shape
    d = x_ref.shape[1] // 2

    x = x_ref[...]
    xl = x[:, :d]
    xr = x[:, d:]
    psel = p_ref[...] > 0

    mpw = mpw_ref[...]
    wc = jax.lax.dot(mpw, wa_ref[...], preferred_element_type=jnp.float32)
    va = va_ref[...]
    tl = jnp.tanh(jax.lax.dot(xl, wc, preferred_element_type=jnp.float32))
    tr = jnp.tanh(jax.lax.dot(xr, wc, preferred_element_type=jnp.float32))
    gl = jax.lax.dot(tl, va, preferred_element_type=jnp.float32)
    gr = jax.lax.dot(tr, va, preferred_element_type=jnp.float32)
    g = jnp.where(psel, gr.reshape(bb, ll), gl.reshape(bb, ll))

    mx = jnp.max(g, axis=1, keepdims=True)
    e = jnp.exp(g - mx)
    alpha = e / jnp.sum(e, axis=1, keepdims=True)

    em3l = xl.reshape(bb, ll, d)
    em3r = xr.reshape(bb, ll, d)
    em0 = jnp.where(psel[:, 0:1], em3r[:, 0, :], em3l[:, 0, :])
    gram = jax.lax.dot(mpw, mpw.T, preferred_element_type=jnp.float32)
    q = jax.lax.dot(em0, gram, preferred_element_type=jnp.float32)
    dl = jnp.sum(em3l * q[:, None, :], axis=2)
    dr = jnp.sum(em3r * q[:, None, :], axis=2)
    dots = jnp.where(psel, dr, dl)

    lidx = jax.lax.broadcasted_iota(jnp.int32, (bb, ll), 1)
    mm = jnp.where(lidx > 0, m_ref[...], 0.0)
    num = jnp.sum(alpha * dots * mm, axis=1, keepdims=True) * alpha[:, 0:1]
    den = jnp.sum(mm, axis=1, keepdims=True) + 1e-8
    out_ref[...] = num / den


def _tc_fused(x128, par2, mask, mp_w, W_a, v_a):
    b, ll = par2.shape
    d = mp_w.shape[0]
    bb = 256
    flat = bb * ll
    return pl.pallas_call(
        _fused_body,
        grid=(b // bb,),
        in_specs=[
            pl.BlockSpec((flat, 2 * d), lambda i: (i, 0)),
            pl.BlockSpec((bb, ll), lambda i: (i, 0)),
            pl.BlockSpec((bb, ll), lambda i: (i, 0)),
            pl.BlockSpec((d, d), lambda i: (0, 0)),
            pl.BlockSpec((d, W_a.shape[1]), lambda i: (0, 0)),
            pl.BlockSpec((W_a.shape[1], 1), lambda i: (0, 0)),
        ],
        out_specs=pl.BlockSpec((bb, 1), lambda i: (i, 0)),
        out_shape=jax.ShapeDtypeStruct((b, 1), jnp.float32),
    )(x128, par2, mask, mp_w, W_a, v_a.reshape(-1, 1))


def kernel(inds, mask, table, mp_w, W_a, v_a):
    b, ll = inds.shape
    v, d = table.shape
    inds32 = inds.astype(jnp.int32)
    table2 = _tc_pair_table(table.T)
    x128 = _sc_gather(table2, (inds32 >> 1).reshape(-1))
    scores = _tc_fused(x128, inds32 & 1, mask, mp_w, W_a, v_a)
    return scores.reshape(b)
